# baseline (speedup 1.0000x reference)
import functools

import jax
import jax.numpy as jnp
import numpy as np
from jax import lax
from jax.experimental import pallas as pl
from jax.experimental.pallas import tpu as pltpu
from jax.experimental.pallas import tpu_sc as plsc

NCLS = 19
SMOOTH_V = 1e-05

NC = 2
NS = 16
L = 16
NW = NC * NS

B, H, W = 4, 512, 512
SC_B = 2
ROWS_PER_W = (SC_B * H) // NW
CHUNK = 8
NCHUNK = ROWS_PER_W // CHUNK
GRP = 8
NACC = 4
ACC = NCLS * L

EXTRA_ROWS = 8
TC_BLK = 256
TC_ROW0 = SC_B * H + NW * EXTRA_ROWS
TC_STEPS = (B * H - TC_ROW0) // TC_BLK


def _sc_body(o_hbm, g_hbm, part_hbm, o_v, g_v, o_x, g_x, sem_o, sem_g, *accs):
    acc_a = accs[:NACC]
    acc_b = accs[NACC:]
    wid = lax.axis_index("s") * NC + lax.axis_index("c")
    wpb = NW // SC_B
    b = wid // wpb
    r0 = (wid % wpb) * ROWS_PER_W

    def start(c, slot):
        ro = r0 + c * CHUNK
        co = pltpu.async_copy(
            o_hbm.at[b, 0, pl.ds(ro, CHUNK), :], o_v.at[slot], sem_o.at[slot]
        )
        cg = pltpu.async_copy(
            g_hbm.at[b, 0, pl.ds(ro, CHUNK), :], g_v.at[slot], sem_g.at[slot]
        )
        return co, cg

    pend = start(0, 0)
    rx = wid * EXTRA_ROWS
    cox = pltpu.async_copy(
        o_hbm.at[SC_B, 0, pl.ds(rx, EXTRA_ROWS), :], o_x, sem_o.at[2]
    )
    cgx = pltpu.async_copy(
        g_hbm.at[SC_B, 0, pl.ds(rx, EXTRA_ROWS), :], g_x, sem_g.at[2]
    )

    zeros = jnp.zeros((L,), jnp.float32)
    for a in accs:
        for r in range(ACC // L):
            a[pl.ds(r * L, L)] = zeros

    lane = lax.iota(jnp.int32, L)
    ones = jnp.ones((L,), jnp.float32)

    def walk(load_g, load_o, nrows):
        def row_body(r, carry):
            for grp in range(W // (L * GRP)):
                gs, os_ = [], []
                for j in range(GRP):
                    col = (grp * GRP + j) * L
                    gs.append(load_g(r, col))
                    os_.append(load_o(r, col))
                d2s, idxs = [], []
                for j in range(GRP):
                    d = gs[j].astype(jnp.float32) - os_[j]
                    d2s.append(d * d)
                    idxs.append(gs[j] * L + lane)
                for j in range(GRP):
                    plsc.addupdate_scatter(acc_a[j % NACC], [idxs[j]], d2s[j])
                    plsc.addupdate_scatter(acc_b[j % NACC], [idxs[j]], ones)
            return carry

        lax.fori_loop(0, nrows, row_body, 0)

    for c in range(NCHUNK):
        co, cg = pend
        co.wait()
        cg.wait()
        if c + 1 < NCHUNK:
            pend = start(c + 1, (c + 1) % 2)
        slot = c % 2
        walk(
            lambda r, col: g_v[slot, r, pl.ds(col, L)],
            lambda r, col: o_v[slot, r, pl.ds(col, L)],
            CHUNK,
        )

    cox.wait()
    cgx.wait()
    walk(
        lambda r, col: g_x[r, pl.ds(col, L)],
        lambda r, col: o_x[r, pl.ds(col, L)],
        EXTRA_ROWS,
    )

    for r in range(ACC // L):
        sl = pl.ds(r * L, L)
        acc_a[0][sl] += acc_a[1][sl] + acc_a[2][sl] + acc_a[3][sl]
        acc_b[0][sl] += acc_b[1][sl] + acc_b[2][sl] + acc_b[3][sl]
    pltpu.sync_copy(acc_a[0], part_hbm.at[wid, 0])
    pltpu.sync_copy(acc_b[0], part_hbm.at[wid, 1])


def _tc_body(o_ref, g_ref, s152_ref, d2_ref, cnt_ref, d2v_ref, ctv_ref):
    i = pl.program_id(0)

    @pl.when(i == 0)
    def _():
        d2_ref[...] = jnp.zeros((NCLS * 8, W), jnp.float32)
        cnt_ref[...] = jnp.zeros((NCLS * 8, W), jnp.float32)

    g = g_ref[0, 0]
    o = o_ref[0, 0]
    d = g.astype(jnp.float32) - o
    d2 = d * d
    for c in range(NCLS):
        m = g == c
        s8 = jnp.sum(jnp.where(m, d2, 0.0).reshape(TC_BLK // 8, 8, W), axis=0)
        n8 = jnp.sum(jnp.where(m, 1.0, 0.0).reshape(TC_BLK // 8, 8, W), axis=0)
        d2_ref[pl.ds(c * 8, 8), :] += s8
        cnt_ref[pl.ds(c * 8, 8), :] += n8

    @pl.when(i == TC_STEPS - 1)
    def _():
        dims = (((1,), (0,)), ((), ()))
        hi = lax.Precision.HIGHEST
        t1 = lax.dot_general(
            s152_ref[...], d2_ref[...], dims, precision=hi,
            preferred_element_type=jnp.float32,
        )
        t2 = lax.dot_general(
            s152_ref[...], cnt_ref[...], dims, precision=hi,
            preferred_element_type=jnp.float32,
        )
        d2v_ref[...] = jnp.sum(t1, axis=1)
        ctv_ref[...] = jnp.sum(t2, axis=1)


def _fin_body(part_ref, d2v_ref, ctv_ref, s304_ref, out_ref):
    dims = (((1,), (0,)), ((), ()))
    hi = lax.Precision.HIGHEST
    total = jnp.sum(part_ref[...], axis=0)
    a = lax.dot_general(
        total, s304_ref[...], dims, precision=hi,
        preferred_element_type=jnp.float32,
    )
    d2 = a[0] + d2v_ref[...]
    ct = a[1] + ctv_ref[...]
    out_ref[...] = d2 / jnp.maximum(ct, SMOOTH_V)


@jax.jit
def _both_call(o, g):
    sc_k = functools.partial(
        pl.kernel,
        out_type=jax.ShapeDtypeStruct((NW, 2, ACC), jnp.float32),
        mesh=plsc.VectorSubcoreMesh(core_axis_name="c", subcore_axis_name="s"),
        compiler_params=pltpu.CompilerParams(needs_layout_passes=False),
        scratch_types=[
            pltpu.VMEM((2, CHUNK, W), jnp.float32),
            pltpu.VMEM((2, CHUNK, W), jnp.int32),
            pltpu.VMEM((EXTRA_ROWS, W), jnp.float32),
            pltpu.VMEM((EXTRA_ROWS, W), jnp.int32),
            pltpu.SemaphoreType.DMA((3,)),
            pltpu.SemaphoreType.DMA((3,)),
        ]
        + [pltpu.VMEM((ACC,), jnp.float32) for _ in range(2 * NACC)],
    )(_sc_body)
    part = sc_k(o, g)

    s304 = jnp.asarray(
        np.eye(NCLS, dtype=np.float32)[np.arange(ACC) // L]
    )
    s152 = jnp.asarray(
        np.eye(NCLS, dtype=np.float32)[np.arange(NCLS * 8) // 8].T
    )

    blk0 = TC_ROW0 // TC_BLK
    rows_per_img = H // TC_BLK
    _, _, tc_d2v, tc_ctv = pl.pallas_call(
        _tc_body,
        grid=(TC_STEPS,),
        in_specs=[
            pl.BlockSpec(
                (1, 1, TC_BLK, W),
                lambda i: ((blk0 + i) // rows_per_img, 0, (blk0 + i) % rows_per_img, 0),
            ),
            pl.BlockSpec(
                (1, 1, TC_BLK, W),
                lambda i: ((blk0 + i) // rows_per_img, 0, (blk0 + i) % rows_per_img, 0),
            ),
            pl.BlockSpec((NCLS, NCLS * 8), lambda i: (0, 0)),
        ],
        out_specs=[
            pl.BlockSpec((NCLS * 8, W), lambda i: (0, 0)),
            pl.BlockSpec((NCLS * 8, W), lambda i: (0, 0)),
            pl.BlockSpec((NCLS,), lambda i: (0,)),
            pl.BlockSpec((NCLS,), lambda i: (0,)),
        ],
        out_shape=[
            jax.ShapeDtypeStruct((NCLS * 8, W), jnp.float32),
            jax.ShapeDtypeStruct((NCLS * 8, W), jnp.float32),
            jax.ShapeDtypeStruct((NCLS,), jnp.float32),
            jax.ShapeDtypeStruct((NCLS,), jnp.float32),
        ],
    )(o, g, s152)

    mse = pl.pallas_call(
        _fin_body,
        out_shape=jax.ShapeDtypeStruct((NCLS,), jnp.float32),
    )(part, tc_d2v, tc_ctv, s304)
    return mse


def kernel(outputs, gt):
    return _both_call(outputs, gt)


# speedup vs baseline: 1.0318x; 1.0318x over previous
import functools

import jax
import jax.numpy as jnp
import numpy as np
from jax import lax
from jax.experimental import pallas as pl
from jax.experimental.pallas import tpu as pltpu
from jax.experimental.pallas import tpu_sc as plsc

NCLS = 19
SMOOTH_V = 1e-05

NC = 2
NS = 16
L = 16
NW = NC * NS

B, H, W = 4, 512, 512
SC_B = 2
ROWS_PER_W = (SC_B * H) // NW
CHUNK = 16
NCHUNK = ROWS_PER_W // CHUNK
GRP = 8
NACC = 4
ACC = NCLS * L

EXTRA_ROWS = 8
TC_BLK = 256
TC_ROW0 = SC_B * H + NW * EXTRA_ROWS
TC_STEPS = (B * H - TC_ROW0) // TC_BLK


def _sc_body(o_hbm, g_hbm, part_hbm, o_v, g_v, o_x, g_x, sem_o, sem_g, *accs):
    acc_a = accs[:NACC]
    acc_b = accs[NACC:]
    wid = lax.axis_index("s") * NC + lax.axis_index("c")
    wpb = NW // SC_B
    b = wid // wpb
    r0 = (wid % wpb) * ROWS_PER_W

    def start(c, slot):
        ro = r0 + c * CHUNK
        co = pltpu.async_copy(
            o_hbm.at[b, 0, pl.ds(ro, CHUNK), :], o_v.at[slot], sem_o.at[slot]
        )
        cg = pltpu.async_copy(
            g_hbm.at[b, 0, pl.ds(ro, CHUNK), :], g_v.at[slot], sem_g.at[slot]
        )
        return co, cg

    pend = start(0, 0)
    rx = wid * EXTRA_ROWS
    cox = pltpu.async_copy(
        o_hbm.at[SC_B, 0, pl.ds(rx, EXTRA_ROWS), :], o_x, sem_o.at[2]
    )
    cgx = pltpu.async_copy(
        g_hbm.at[SC_B, 0, pl.ds(rx, EXTRA_ROWS), :], g_x, sem_g.at[2]
    )

    zeros = jnp.zeros((L,), jnp.float32)
    for a in accs:
        for r in range(ACC // L):
            a[pl.ds(r * L, L)] = zeros

    lane = lax.iota(jnp.int32, L)
    ones = jnp.ones((L,), jnp.float32)

    def walk(load_g, load_o, nrows):
        def row_body(r, carry):
            for grp in range(W // (L * GRP)):
                gs, os_ = [], []
                for j in range(GRP):
                    col = (grp * GRP + j) * L
                    gs.append(load_g(r, col))
                    os_.append(load_o(r, col))
                d2s, idxs = [], []
                for j in range(GRP):
                    d = gs[j].astype(jnp.float32) - os_[j]
                    d2s.append(d * d)
                    idxs.append(gs[j] * L + lane)
                for j in range(GRP):
                    plsc.addupdate_scatter(acc_a[j % NACC], [idxs[j]], d2s[j])
                    plsc.addupdate_scatter(acc_b[j % NACC], [idxs[j]], ones)
            return carry

        lax.fori_loop(0, nrows, row_body, 0)

    for c in range(NCHUNK):
        co, cg = pend
        co.wait()
        cg.wait()
        if c + 1 < NCHUNK:
            pend = start(c + 1, (c + 1) % 2)
        slot = c % 2
        walk(
            lambda r, col: g_v[slot, r, pl.ds(col, L)],
            lambda r, col: o_v[slot, r, pl.ds(col, L)],
            CHUNK,
        )

    cox.wait()
    cgx.wait()
    walk(
        lambda r, col: g_x[r, pl.ds(col, L)],
        lambda r, col: o_x[r, pl.ds(col, L)],
        EXTRA_ROWS,
    )

    for r in range(ACC // L):
        sl = pl.ds(r * L, L)
        acc_a[0][sl] += acc_a[1][sl] + acc_a[2][sl] + acc_a[3][sl]
        acc_b[0][sl] += acc_b[1][sl] + acc_b[2][sl] + acc_b[3][sl]
    pltpu.sync_copy(acc_a[0], part_hbm.at[wid, 0])
    pltpu.sync_copy(acc_b[0], part_hbm.at[wid, 1])


def _tc_body(o_ref, g_ref, s152_ref, d2_ref, cnt_ref, d2v_ref, ctv_ref):
    i = pl.program_id(0)

    @pl.when(i == 0)
    def _():
        d2_ref[...] = jnp.zeros((NCLS * 8, W), jnp.float32)
        cnt_ref[...] = jnp.zeros((NCLS * 8, W), jnp.float32)

    g = g_ref[0, 0]
    o = o_ref[0, 0]
    d = g.astype(jnp.float32) - o
    d2 = d * d
    for c in range(NCLS):
        m = g == c
        s8 = jnp.sum(jnp.where(m, d2, 0.0).reshape(TC_BLK // 8, 8, W), axis=0)
        n8 = jnp.sum(jnp.where(m, 1.0, 0.0).reshape(TC_BLK // 8, 8, W), axis=0)
        d2_ref[pl.ds(c * 8, 8), :] += s8
        cnt_ref[pl.ds(c * 8, 8), :] += n8

    @pl.when(i == TC_STEPS - 1)
    def _():
        dims = (((1,), (0,)), ((), ()))
        hi = lax.Precision.HIGHEST
        t1 = lax.dot_general(
            s152_ref[...], d2_ref[...], dims, precision=hi,
            preferred_element_type=jnp.float32,
        )
        t2 = lax.dot_general(
            s152_ref[...], cnt_ref[...], dims, precision=hi,
            preferred_element_type=jnp.float32,
        )
        d2v_ref[...] = jnp.sum(t1, axis=1)
        ctv_ref[...] = jnp.sum(t2, axis=1)


def _fin_body(part_ref, d2v_ref, ctv_ref, s304_ref, out_ref):
    dims = (((1,), (0,)), ((), ()))
    hi = lax.Precision.HIGHEST
    total = jnp.sum(part_ref[...], axis=0)
    a = lax.dot_general(
        total, s304_ref[...], dims, precision=hi,
        preferred_element_type=jnp.float32,
    )
    d2 = a[0] + d2v_ref[...]
    ct = a[1] + ctv_ref[...]
    out_ref[...] = d2 / jnp.maximum(ct, SMOOTH_V)


@jax.jit
def _both_call(o, g):
    sc_k = functools.partial(
        pl.kernel,
        out_type=jax.ShapeDtypeStruct((NW, 2, ACC), jnp.float32),
        mesh=plsc.VectorSubcoreMesh(core_axis_name="c", subcore_axis_name="s"),
        compiler_params=pltpu.CompilerParams(needs_layout_passes=False),
        scratch_types=[
            pltpu.VMEM((2, CHUNK, W), jnp.float32),
            pltpu.VMEM((2, CHUNK, W), jnp.int32),
            pltpu.VMEM((EXTRA_ROWS, W), jnp.float32),
            pltpu.VMEM((EXTRA_ROWS, W), jnp.int32),
            pltpu.SemaphoreType.DMA((3,)),
            pltpu.SemaphoreType.DMA((3,)),
        ]
        + [pltpu.VMEM((ACC,), jnp.float32) for _ in range(2 * NACC)],
    )(_sc_body)
    part = sc_k(o, g)

    s304 = jnp.asarray(
        np.eye(NCLS, dtype=np.float32)[np.arange(ACC) // L]
    )
    s152 = jnp.asarray(
        np.eye(NCLS, dtype=np.float32)[np.arange(NCLS * 8) // 8].T
    )

    blk0 = TC_ROW0 // TC_BLK
    rows_per_img = H // TC_BLK
    _, _, tc_d2v, tc_ctv = pl.pallas_call(
        _tc_body,
        grid=(TC_STEPS,),
        in_specs=[
            pl.BlockSpec(
                (1, 1, TC_BLK, W),
                lambda i: ((blk0 + i) // rows_per_img, 0, (blk0 + i) % rows_per_img, 0),
            ),
            pl.BlockSpec(
                (1, 1, TC_BLK, W),
                lambda i: ((blk0 + i) // rows_per_img, 0, (blk0 + i) % rows_per_img, 0),
            ),
            pl.BlockSpec((NCLS, NCLS * 8), lambda i: (0, 0)),
        ],
        out_specs=[
            pl.BlockSpec((NCLS * 8, W), lambda i: (0, 0)),
            pl.BlockSpec((NCLS * 8, W), lambda i: (0, 0)),
            pl.BlockSpec((NCLS,), lambda i: (0,)),
            pl.BlockSpec((NCLS,), lambda i: (0,)),
        ],
        out_shape=[
            jax.ShapeDtypeStruct((NCLS * 8, W), jnp.float32),
            jax.ShapeDtypeStruct((NCLS * 8, W), jnp.float32),
            jax.ShapeDtypeStruct((NCLS,), jnp.float32),
            jax.ShapeDtypeStruct((NCLS,), jnp.float32),
        ],
    )(o, g, s152)

    mse = pl.pallas_call(
        _fin_body,
        out_shape=jax.ShapeDtypeStruct((NCLS,), jnp.float32),
    )(part, tc_d2v, tc_ctv, s304)
    return mse


def kernel(outputs, gt):
    return _both_call(outputs, gt)
